# Initial kernel scaffold; baseline (speedup 1.0000x reference)
import functools
import math

import jax
import jax.numpy as jnp
from jax import lax
from jax.experimental import pallas as pl
from jax.experimental.pallas import tpu as pltpu
from jax.experimental.pallas import tpu_sc as plsc

GRID_H, GRID_W, FEAT = 1024, 1024, 32
PI = math.pi

NW = 32
B = 128
L = 16

X_SCALE = (GRID_H - 1.0) / PI
Y_SCALE = (GRID_W - 1.0) / (2.0 * PI)
Y_OFF = (GRID_W - 1.0) / 2.0


def _sc_gather_combine(n_pad):
    kpw = n_pad // (NW * B)
    mesh = plsc.VectorSubcoreMesh(core_axis_name="c", subcore_axis_name="s")

    @functools.partial(
        pl.kernel,
        mesh=mesh,
        out_type=jax.ShapeDtypeStruct((n_pad, 2 * FEAT), jnp.float32),
        scratch_types=[
            pltpu.VMEM((4, B), jnp.float32),
            pltpu.VMEM((4, B), jnp.int32),
            pltpu.VMEM((4, B), jnp.int32),
            pltpu.VMEM((4, B), jnp.float32),
            pltpu.VMEM((4, B), jnp.float32),
            pltpu.VMEM((4, B, FEAT), jnp.float32),
            pltpu.VMEM((4, B, FEAT), jnp.float32),
            pltpu.VMEM((B, 2 * FEAT), jnp.float32),
            pltpu.SemaphoreType.DMA,
        ],
    )
    def sc_kernel(coords_hbm, gp_hbm, gd_hbm, out_hbm,
                  cv, ipos, idir, wpos, wdir, gpos, gdir, fv, gsem):
        wid = lax.axis_index("s") * 2 + lax.axis_index("c")

        def block_body(t, _):
            g = wid * kpw + t
            pltpu.sync_copy(coords_hbm.at[g], cv)

            for j in range(B // L):
                s = pl.ds(j * L, L)
                for grid_sel in range(2):
                    vx = cv[2 * grid_sel, s]
                    vy = cv[2 * grid_sel + 1, s]
                    xs = vx * X_SCALE
                    ys = vy * Y_SCALE + Y_OFF
                    tlx = xs.astype(jnp.int32)
                    tly = ys.astype(jnp.int32)
                    xf = xs - tlx.astype(jnp.float32)
                    yf = ys - tly.astype(jnp.float32)
                    brx = jnp.minimum(tlx + 1, GRID_W - 1)
                    bry = jnp.minimum(tly + 1, GRID_H - 1)
                    rtop = tly * GRID_W
                    rbot = bry * GRID_W
                    irefs = ipos if grid_sel == 0 else idir
                    wrefs = wpos if grid_sel == 0 else wdir
                    irefs[0, s] = rtop + tlx
                    irefs[1, s] = rtop + brx
                    irefs[2, s] = rbot + tlx
                    irefs[3, s] = rbot + brx
                    omx = 1.0 - xf
                    omy = 1.0 - yf
                    wrefs[0, s] = omy * omx
                    wrefs[1, s] = omy * xf
                    wrefs[2, s] = yf * omx
                    wrefs[3, s] = yf * xf

            copies = []
            for c in range(4):
                copies.append(pltpu.async_copy(gp_hbm.at[ipos.at[c]], gpos.at[c], gsem))
                copies.append(pltpu.async_copy(gd_hbm.at[idir.at[c]], gdir.at[c], gsem))
            for cp in copies:
                cp.wait()

            def combine(q, _):
                w0 = wpos[0, q]
                w1 = wpos[1, q]
                w2 = wpos[2, q]
                w3 = wpos[3, q]
                u0 = wdir[0, q]
                u1 = wdir[1, q]
                u2 = wdir[2, q]
                u3 = wdir[3, q]
                for h in range(FEAT // L):
                    sl = pl.ds(h * L, L)
                    accp = (gpos[0, q, sl] * w0 + gpos[1, q, sl] * w1
                            + gpos[2, q, sl] * w2 + gpos[3, q, sl] * w3)
                    fv[q, pl.ds(h * L, L)] = accp
                    accd = (gdir[0, q, sl] * u0 + gdir[1, q, sl] * u1
                            + gdir[2, q, sl] * u2 + gdir[3, q, sl] * u3)
                    fv[q, pl.ds(FEAT + h * L, L)] = accd
                return 0

            lax.fori_loop(0, B, combine, 0)
            pltpu.sync_copy(fv, out_hbm.at[pl.ds(g * B, B)])
            return 0

        lax.fori_loop(0, kpw, block_body, 0)

    return sc_kernel


def _mlp_body(x_ref, w1_ref, b1_ref, w2_ref, b2_ref, o_ref):
    x = x_ref[...]
    h = jnp.dot(x, w1_ref[...], preferred_element_type=jnp.float32) + b1_ref[...]
    h = jnp.where(h > 0, h, h * 0.01)
    o = jnp.dot(h, w2_ref[...], preferred_element_type=jnp.float32) + b2_ref[...]
    o_ref[...] = jax.nn.sigmoid(o) * 255.0


def _mlp(feats, W1, b1, W2, b2, blk_m):
    n_pad = feats.shape[0]
    return pl.pallas_call(
        _mlp_body,
        grid=(n_pad // blk_m,),
        in_specs=[
            pl.BlockSpec((blk_m, 2 * FEAT), lambda i: (i, 0)),
            pl.BlockSpec((2 * FEAT, 4 * FEAT), lambda i: (0, 0)),
            pl.BlockSpec((1, 4 * FEAT), lambda i: (0, 0)),
            pl.BlockSpec((4 * FEAT, 3), lambda i: (0, 0)),
            pl.BlockSpec((1, 3), lambda i: (0, 0)),
        ],
        out_specs=pl.BlockSpec((blk_m, 3), lambda i: (i, 0)),
        out_shape=jax.ShapeDtypeStruct((n_pad, 3), jnp.float32),
    )(feats, W1, b1.reshape(1, -1), W2, b2.reshape(1, -1))


def kernel(pos, dir, grid_pos, grid_dir, W1, b1, W2, b2):
    n = pos.shape[0]
    kpw = -(-n // (NW * B))
    n_pad = NW * B * kpw

    coords = jnp.concatenate([pos.T, dir.T], axis=0)
    coords = jnp.pad(coords, ((0, 0), (0, n_pad - n)))
    coords = coords.reshape(4, n_pad // B, B).transpose(1, 0, 2)
    gp = grid_pos.reshape(GRID_H * GRID_W, FEAT)
    gd = grid_dir.reshape(GRID_H * GRID_W, FEAT)

    feats = _sc_gather_combine(n_pad)(coords, gp, gd)
    out = _mlp(feats, W1, b1, W2, b2, blk_m=2048)
    return out[:n]


# speedup vs baseline: 2.6927x; 2.6927x over previous
import functools
import math

import jax
import jax.numpy as jnp
from jax import lax
from jax.experimental import pallas as pl
from jax.experimental.pallas import tpu as pltpu
from jax.experimental.pallas import tpu_sc as plsc

GRID_H, GRID_W, FEAT = 1024, 1024, 32
PI = math.pi

NW = 32
B = 128
L = 16

X_SCALE = (GRID_H - 1.0) / PI
Y_SCALE = (GRID_W - 1.0) / (2.0 * PI)
Y_OFF = (GRID_W - 1.0) / 2.0


def _sc_gather_combine(n_pad):
    kpw = n_pad // (NW * B)
    mesh = plsc.VectorSubcoreMesh(core_axis_name="c", subcore_axis_name="s")

    @functools.partial(
        pl.kernel,
        mesh=mesh,
        compiler_params=pltpu.CompilerParams(use_tc_tiling_on_sc=False),
        out_type=jax.ShapeDtypeStruct((n_pad, 2 * FEAT), jnp.float32),
        scratch_types=[
            pltpu.VMEM((4, B), jnp.float32),
            pltpu.VMEM((4, B), jnp.int32),
            pltpu.VMEM((4, B), jnp.int32),
            pltpu.VMEM((4, B), jnp.float32),
            pltpu.VMEM((4, B), jnp.float32),
            pltpu.VMEM((4, B, FEAT), jnp.float32),
            pltpu.VMEM((4, B, FEAT), jnp.float32),
            pltpu.VMEM((B, 2 * FEAT), jnp.float32),
            pltpu.SemaphoreType.DMA,
        ],
    )
    def sc_kernel(coords_hbm, gp_hbm, gd_hbm, out_hbm,
                  cv, ipos, idir, wpos, wdir, gpos, gdir, fv, gsem):
        wid = lax.axis_index("s") * 2 + lax.axis_index("c")

        def block_body(t, _):
            g = wid * kpw + t
            pltpu.sync_copy(coords_hbm.at[g], cv)

            for j in range(B // L):
                s = pl.ds(j * L, L)
                for grid_sel in range(2):
                    vx = cv[2 * grid_sel, s]
                    vy = cv[2 * grid_sel + 1, s]
                    xs = vx * X_SCALE
                    ys = vy * Y_SCALE + Y_OFF
                    tlx = xs.astype(jnp.int32)
                    tly = ys.astype(jnp.int32)
                    xf = xs - tlx.astype(jnp.float32)
                    yf = ys - tly.astype(jnp.float32)
                    brx = jnp.minimum(tlx + 1, GRID_W - 1)
                    bry = jnp.minimum(tly + 1, GRID_H - 1)
                    rtop = tly * GRID_W
                    rbot = bry * GRID_W
                    irefs = ipos if grid_sel == 0 else idir
                    wrefs = wpos if grid_sel == 0 else wdir
                    irefs[0, s] = rtop + tlx
                    irefs[1, s] = rtop + brx
                    irefs[2, s] = rbot + tlx
                    irefs[3, s] = rbot + brx
                    omx = 1.0 - xf
                    omy = 1.0 - yf
                    wrefs[0, s] = omy * omx
                    wrefs[1, s] = omy * xf
                    wrefs[2, s] = yf * omx
                    wrefs[3, s] = yf * xf

            copies = []
            for c in range(4):
                copies.append(pltpu.async_copy(gp_hbm.at[ipos.at[c]], gpos.at[c], gsem))
                copies.append(pltpu.async_copy(gd_hbm.at[idir.at[c]], gdir.at[c], gsem))
            for cp in copies:
                cp.wait()

            def combine(jg, _):
                base = jg * L
                wv = [wpos[c, pl.ds(base, L)] for c in range(4)]
                uv = [wdir[c, pl.ds(base, L)] for c in range(4)]
                for qi in range(L):
                    q = base + qi
                    for h in range(FEAT // L):
                        sl = pl.ds(h * L, L)
                        accp = (gpos[0, q, sl] * wv[0][qi]
                                + gpos[1, q, sl] * wv[1][qi]
                                + gpos[2, q, sl] * wv[2][qi]
                                + gpos[3, q, sl] * wv[3][qi])
                        fv[q, pl.ds(h * L, L)] = accp
                        accd = (gdir[0, q, sl] * uv[0][qi]
                                + gdir[1, q, sl] * uv[1][qi]
                                + gdir[2, q, sl] * uv[2][qi]
                                + gdir[3, q, sl] * uv[3][qi])
                        fv[q, pl.ds(FEAT + h * L, L)] = accd
                return 0

            lax.fori_loop(0, B // L, combine, 0)
            pltpu.sync_copy(fv, out_hbm.at[pl.ds(g * B, B)])
            return 0

        lax.fori_loop(0, kpw, block_body, 0)

    return sc_kernel


def _mlp_body(x_ref, w1_ref, b1_ref, w2_ref, b2_ref, o_ref):
    x = x_ref[...]
    h = jnp.dot(x, w1_ref[...], preferred_element_type=jnp.float32) + b1_ref[...]
    h = jnp.where(h > 0, h, h * 0.01)
    o = jnp.dot(h, w2_ref[...], preferred_element_type=jnp.float32) + b2_ref[...]
    o_ref[...] = jax.nn.sigmoid(o) * 255.0


def _mlp(feats, W1, b1, W2, b2, blk_m):
    n_pad = feats.shape[0]
    return pl.pallas_call(
        _mlp_body,
        grid=(n_pad // blk_m,),
        in_specs=[
            pl.BlockSpec((blk_m, 2 * FEAT), lambda i: (i, 0)),
            pl.BlockSpec((2 * FEAT, 4 * FEAT), lambda i: (0, 0)),
            pl.BlockSpec((1, 4 * FEAT), lambda i: (0, 0)),
            pl.BlockSpec((4 * FEAT, 3), lambda i: (0, 0)),
            pl.BlockSpec((1, 3), lambda i: (0, 0)),
        ],
        out_specs=pl.BlockSpec((blk_m, 3), lambda i: (i, 0)),
        out_shape=jax.ShapeDtypeStruct((n_pad, 3), jnp.float32),
    )(feats, W1, b1.reshape(1, -1), W2, b2.reshape(1, -1))


def kernel(pos, dir, grid_pos, grid_dir, W1, b1, W2, b2):
    n = pos.shape[0]
    kpw = -(-n // (NW * B))
    n_pad = NW * B * kpw

    coords = jnp.concatenate([pos.T, dir.T], axis=0)
    coords = jnp.pad(coords, ((0, 0), (0, n_pad - n)))
    coords = coords.reshape(4, n_pad // B, B).transpose(1, 0, 2)
    gp = grid_pos.reshape(GRID_H * GRID_W, FEAT)
    gd = grid_dir.reshape(GRID_H * GRID_W, FEAT)

    feats = _sc_gather_combine(n_pad)(coords, gp, gd)
    out = _mlp(feats, W1, b1, W2, b2, blk_m=2048)
    return out[:n]


# speedup vs baseline: 3.0629x; 1.1375x over previous
import functools
import math

import jax
import jax.numpy as jnp
from jax import lax
from jax.experimental import pallas as pl
from jax.experimental.pallas import tpu as pltpu
from jax.experimental.pallas import tpu_sc as plsc

GRID_H, GRID_W, FEAT = 1024, 1024, 32
PI = math.pi

NW = 32
B = 128
L = 16

X_SCALE = (GRID_H - 1.0) / PI
Y_SCALE = (GRID_W - 1.0) / (2.0 * PI)
Y_OFF = (GRID_W - 1.0) / 2.0


def _sc_gather_combine(n_pad):
    kpw = n_pad // (NW * B)
    assert kpw % 2 == 0
    mesh = plsc.VectorSubcoreMesh(core_axis_name="c", subcore_axis_name="s")

    @functools.partial(
        pl.kernel,
        mesh=mesh,
        compiler_params=pltpu.CompilerParams(use_tc_tiling_on_sc=False),
        out_type=jax.ShapeDtypeStruct((n_pad, 2 * FEAT), jnp.float32),
        scratch_types=[
            pltpu.VMEM((4, B), jnp.float32),
            pltpu.VMEM((2, 4, B), jnp.int32),
            pltpu.VMEM((2, 4, B), jnp.int32),
            pltpu.VMEM((2, 4, B), jnp.float32),
            pltpu.VMEM((2, 4, B), jnp.float32),
            pltpu.VMEM((2, 4, B, FEAT), jnp.float32),
            pltpu.VMEM((2, 4, B, FEAT), jnp.float32),
            pltpu.VMEM((2, B, 2 * FEAT), jnp.float32),
            pltpu.SemaphoreType.DMA,
            pltpu.SemaphoreType.DMA,
            pltpu.SemaphoreType.DMA,
            pltpu.SemaphoreType.DMA,
        ],
    )
    def sc_kernel(coords_hbm, gp_hbm, gd_hbm, out_hbm,
                  cv, ipos, idir, wpos, wdir, gpos, gdir, fv,
                  gsem0, gsem1, osem0, osem1):
        wid = lax.axis_index("s") * 2 + lax.axis_index("c")
        wbase = wid * kpw
        gsems = [gsem0, gsem1]
        osems = [osem0, osem1]

        def prep(p, g):
            pltpu.sync_copy(coords_hbm.at[g], cv)
            for j in range(B // L):
                s = pl.ds(j * L, L)
                for grid_sel in range(2):
                    vx = cv[2 * grid_sel, s]
                    vy = cv[2 * grid_sel + 1, s]
                    xs = vx * X_SCALE
                    ys = vy * Y_SCALE + Y_OFF
                    tlx = xs.astype(jnp.int32)
                    tly = ys.astype(jnp.int32)
                    xf = xs - tlx.astype(jnp.float32)
                    yf = ys - tly.astype(jnp.float32)
                    brx = jnp.minimum(tlx + 1, GRID_W - 1)
                    bry = jnp.minimum(tly + 1, GRID_H - 1)
                    rtop = tly * GRID_W
                    rbot = bry * GRID_W
                    irefs = ipos if grid_sel == 0 else idir
                    wrefs = wpos if grid_sel == 0 else wdir
                    irefs[p, 0, s] = rtop + tlx
                    irefs[p, 1, s] = rtop + brx
                    irefs[p, 2, s] = rbot + tlx
                    irefs[p, 3, s] = rbot + brx
                    omx = 1.0 - xf
                    omy = 1.0 - yf
                    wrefs[p, 0, s] = omy * omx
                    wrefs[p, 1, s] = omy * xf
                    wrefs[p, 2, s] = yf * omx
                    wrefs[p, 3, s] = yf * xf
            for c in range(4):
                pltpu.async_copy(gp_hbm.at[ipos.at[p, c]], gpos.at[p, c], gsems[p])
                pltpu.async_copy(gd_hbm.at[idir.at[p, c]], gdir.at[p, c], gsems[p])

        def finish(p, t, g):
            for c in range(4):
                pltpu.make_async_copy(gp_hbm.at[ipos.at[p, c]], gpos.at[p, c], gsems[p]).wait()
                pltpu.make_async_copy(gd_hbm.at[idir.at[p, c]], gdir.at[p, c], gsems[p]).wait()

            @pl.when(t >= 2)
            def _():
                pltpu.make_async_copy(fv.at[p], out_hbm.at[pl.ds(g * B, B)], osems[p]).wait()

            def combine(jg, _):
                base = jg * L
                wv = [wpos[p, c, pl.ds(base, L)] for c in range(4)]
                uv = [wdir[p, c, pl.ds(base, L)] for c in range(4)]
                for qi in range(L):
                    q = base + qi
                    for h in range(FEAT // L):
                        sl = pl.ds(h * L, L)
                        accp = (gpos[p, 0, q, sl] * wv[0][qi]
                                + gpos[p, 1, q, sl] * wv[1][qi]
                                + gpos[p, 2, q, sl] * wv[2][qi]
                                + gpos[p, 3, q, sl] * wv[3][qi])
                        fv[p, q, pl.ds(h * L, L)] = accp
                        accd = (gdir[p, 0, q, sl] * uv[0][qi]
                                + gdir[p, 1, q, sl] * uv[1][qi]
                                + gdir[p, 2, q, sl] * uv[2][qi]
                                + gdir[p, 3, q, sl] * uv[3][qi])
                        fv[p, q, pl.ds(FEAT + h * L, L)] = accd
                return 0

            lax.fori_loop(0, B // L, combine, 0)
            pltpu.async_copy(fv.at[p], out_hbm.at[pl.ds(g * B, B)], osems[p])

        prep(0, wbase)

        def body(t2, _):
            t0 = 2 * t2
            g0 = wbase + t0
            prep(1, g0 + 1)
            finish(0, t0, g0)

            @pl.when(t2 < kpw // 2 - 1)
            def _():
                prep(0, g0 + 2)

            finish(1, t0 + 1, g0 + 1)
            return 0

        lax.fori_loop(0, kpw // 2, body, 0)
        for p in range(2):
            pltpu.make_async_copy(fv.at[p], out_hbm.at[pl.ds(wbase * B, B)], osems[p]).wait()

    return sc_kernel


def _mlp_body(x_ref, w1_ref, b1_ref, w2_ref, b2_ref, o_ref):
    x = x_ref[...]
    h = jnp.dot(x, w1_ref[...], preferred_element_type=jnp.float32) + b1_ref[...]
    h = jnp.where(h > 0, h, h * 0.01)
    o = jnp.dot(h, w2_ref[...], preferred_element_type=jnp.float32) + b2_ref[...]
    o_ref[...] = jax.nn.sigmoid(o) * 255.0


def _mlp(feats, W1, b1, W2, b2, blk_m):
    n_pad = feats.shape[0]
    return pl.pallas_call(
        _mlp_body,
        grid=(n_pad // blk_m,),
        in_specs=[
            pl.BlockSpec((blk_m, 2 * FEAT), lambda i: (i, 0)),
            pl.BlockSpec((2 * FEAT, 4 * FEAT), lambda i: (0, 0)),
            pl.BlockSpec((1, 4 * FEAT), lambda i: (0, 0)),
            pl.BlockSpec((4 * FEAT, 3), lambda i: (0, 0)),
            pl.BlockSpec((1, 3), lambda i: (0, 0)),
        ],
        out_specs=pl.BlockSpec((blk_m, 3), lambda i: (i, 0)),
        out_shape=jax.ShapeDtypeStruct((n_pad, 3), jnp.float32),
    )(feats, W1, b1.reshape(1, -1), W2, b2.reshape(1, -1))


def kernel(pos, dir, grid_pos, grid_dir, W1, b1, W2, b2):
    n = pos.shape[0]
    kpw = -(-n // (NW * B))
    kpw += kpw % 2
    n_pad = NW * B * kpw

    coords = jnp.concatenate([pos.T, dir.T], axis=0)
    coords = jnp.pad(coords, ((0, 0), (0, n_pad - n)))
    coords = coords.reshape(4, n_pad // B, B).transpose(1, 0, 2)
    gp = grid_pos.reshape(GRID_H * GRID_W, FEAT)
    gd = grid_dir.reshape(GRID_H * GRID_W, FEAT)

    feats = _sc_gather_combine(n_pad)(coords, gp, gd)
    out = _mlp(feats, W1, b1, W2, b2, blk_m=2048)
    return out[:n]
